# baseline (speedup 1.0000x reference)
import jax
import jax.numpy as jnp
from jax import lax
from jax.experimental import pallas as pl
from jax.experimental.pallas import tpu as pltpu
from jax.experimental.pallas import tpu_sc as plsc

N = 8192
D = 128
T = 8
E = 24576
EW = 32
SW = 32
NWIN = E // EW
NC = 2
NS = 16
TH = T // 2
TPC = TH // NC
WPT = NWIN // NS
BSZ = 2048
VW = N // SW
VWT = VW // NS


def _ab_body(x_ref, wa_ref, wb_ref, bv_ref, a_ref, b_ref):
    x = x_ref[0]
    a_ref[0] = jnp.dot(x, wa_ref[...], preferred_element_type=jnp.float32)
    b_ref[0] = (jnp.dot(x, wb_ref[...], preferred_element_type=jnp.float32)
                + bv_ref[...])


def _ab(lat_t, wa, wb, bv, t_base=0):
    bn = 2048
    return pl.pallas_call(
        _ab_body,
        grid=(TH, N // bn),
        in_specs=[
            pl.BlockSpec((1, bn, D), lambda t, i: (t + t_base, i, 0)),
            pl.BlockSpec((D, D), lambda t, i: (0, 0)),
            pl.BlockSpec((D, D), lambda t, i: (0, 0)),
            pl.BlockSpec((1, D), lambda t, i: (0, 0)),
        ],
        out_specs=[
            pl.BlockSpec((1, bn, D), lambda t, i: (t, i, 0)),
            pl.BlockSpec((1, bn, D), lambda t, i: (t, i, 0)),
        ],
        out_shape=[
            jax.ShapeDtypeStruct((TH, N, D), jnp.float32),
            jax.ShapeDtypeStruct((TH, N, D), jnp.float32),
        ],
    )(lat_t, wa, wb, bv)


def _readout_body(ya_ref, yb_ref, w0_ref, b0_ref, w1_ref, b1_ref, o_ref):
    for t in range(T):
        y = ya_ref[t] if t < TH else yb_ref[t - TH]
        h = jnp.maximum(
            jnp.dot(y, w0_ref[...], preferred_element_type=jnp.float32)
            + b0_ref[...], 0.0)
        o_ref[:, t, :] = (
            jnp.dot(h, w1_ref[...], preferred_element_type=jnp.float32)
            + b1_ref[...])


def _readout(ya, yb, w0, b0, w1, b1):
    bb = 512
    return pl.pallas_call(
        _readout_body,
        grid=(BSZ // bb,),
        in_specs=[
            pl.BlockSpec((TH, bb, D), lambda i: (0, i, 0)),
            pl.BlockSpec((TH, bb, D), lambda i: (0, i, 0)),
            pl.BlockSpec((D, D), lambda i: (0, 0)),
            pl.BlockSpec((1, D), lambda i: (0, 0)),
            pl.BlockSpec((D, D), lambda i: (0, 0)),
            pl.BlockSpec((1, D), lambda i: (0, 0)),
        ],
        out_specs=pl.BlockSpec((bb, T, D), lambda i: (i, 0, 0)),
        out_shape=jax.ShapeDtypeStruct((BSZ, T, D), jnp.float32),
    )(ya, yb, w0, b0, w1, b1)


_MESH = dict(core_axis_name="c", subcore_axis_name="s", num_cores=NC,
             num_subcores=NS)


def _compute_msg(abuf, bbuf, mbuf):
    def row_body(r, carry):
        for g in range(D // 16):
            sl = pl.ds(g * 16, 16)
            mbuf[r, sl] = jnp.maximum(abuf[r, sl] + bbuf[r, sl], 0.0)
        return carry

    lax.fori_loop(0, EW, row_body, 0, unroll=4)


def _make_issue(a_t, b_t, idxf_all, idxt_all, abufs, bbufs, semas, sembs):
    def issue(w, bi):
        pltpu.async_copy(a_t.at[idxf_all.at[w]], abufs[bi], semas[bi])
        pltpu.async_copy(b_t.at[idxt_all.at[w]], bbufs[bi], sembs[bi])
    return issue


def _edge_windows(t, s, a_hbm, b_hbm, idxf_all, idxt_all, acc,
                  abufs, bbufs, semas, sembs, aux):
    a_t = a_hbm.at[t]
    b_t = b_hbm.at[t]
    mbufs, semss = aux
    issue = _make_issue(a_t, b_t, idxf_all, idxt_all, abufs, bbufs,
                        semas, sembs)

    def wait_gather(bi):
        pltpu.make_async_copy(a_t.at[idxf_all.at[0]], abufs[bi],
                              semas[bi]).wait()
        pltpu.make_async_copy(b_t.at[idxt_all.at[0]], bbufs[bi],
                              sembs[bi]).wait()

    def wait_scatter(mi):
        pltpu.make_async_copy(mbufs[mi], acc.at[idxt_all.at[0]],
                              semss[mi]).wait()

    ntrip = WPT // 6
    issue(1, 1)

    def trip_body(k, carry):
        base = 6 * k
        for j in range(6):
            bi = j % 3
            mi = j % 2
            w = base + j
            if j < 4:
                issue(w + 2, (j + 2) % 3)
            else:
                @pl.when(k < ntrip - 1)
                def _pref():
                    issue(w + 2, (j + 2) % 3)
            wait_gather(bi)
            if j < 2:
                @pl.when(k > 0)
                def _drain():
                    wait_scatter(mi)
            else:
                wait_scatter(mi)
            _compute_msg(abufs[bi], bbufs[bi], mbufs[mi])
            pltpu.async_copy(mbufs[mi], acc.at[idxt_all.at[w]], semss[mi],
                             add=True)
        return carry

    lax.fori_loop(0, ntrip, trip_body, 0)
    wait_scatter(0)
    wait_scatter(1)


def _edge_body_fused(fuse_seg, lat_base, lat_hbm, a_hbm, b_hbm,
                     from_hbm, to_hbm, vid_hbm, zeros_hbm, out_hbm, y_hbm,
                     acc, ysum, idxf_all, idxt_all, idxv,
                     ab0, ab1, ab2, bb0, bb1, bb2, mb0, mb1,
                     sema0, sema1, sema2, semb0, semb1, semb2,
                     sems0, sems1):
    c = lax.axis_index("c")
    s = lax.axis_index("s")
    abufs = (ab0, ab1, ab2)
    bbufs = (bb0, bb1, bb2)
    semas = (sema0, sema1, sema2)
    sembs = (semb0, semb1, semb2)
    aux = ((mb0, mb1), (sems0, sems1))

    pltpu.sync_copy(from_hbm.at[s], idxf_all)
    pltpu.sync_copy(to_hbm.at[s], idxt_all)

    nsl = N // NS
    ysl = BSZ // NS
    for p in range(TPC):
        t = c * TPC + p
        tg = t + lat_base

        issue0 = _make_issue(a_hbm.at[t], b_hbm.at[t], idxf_all, idxt_all,
                             abufs, bbufs, semas, sembs)
        issue0(0, 0)

        aoff = pl.multiple_of(s * nsl, nsl)
        pltpu.sync_copy(lat_hbm.at[tg, pl.ds(aoff, nsl), :],
                        acc.at[pl.ds(aoff, nsl)])
        if fuse_seg:
            yoff = pl.multiple_of(s * ysl, ysl)
            pltpu.sync_copy(zeros_hbm.at[pl.ds(yoff, ysl)],
                            ysum.at[pl.ds(yoff, ysl)])

        plsc.subcore_barrier()

        _edge_windows(t, s, a_hbm, b_hbm, idxf_all, idxt_all, acc,
                      abufs, bbufs, semas, sembs, aux)
        plsc.subcore_barrier()

        if fuse_seg:
            pltpu.sync_copy(vid_hbm.at[s], idxv)
            for w in range(VWT):
                row = s * VWT + w
                off = pl.multiple_of(row * SW, SW)
                pltpu.sync_copy(acc.at[pl.ds(off, SW)], mb0)
                pltpu.sync_copy(mb0, ysum.at[idxv.at[w]], add=True)
            plsc.subcore_barrier()

        if fuse_seg:
            yoff2 = pl.multiple_of(s * ysl, ysl)
            pltpu.sync_copy(ysum.at[pl.ds(yoff2, ysl)],
                            y_hbm.at[t, pl.ds(yoff2, ysl), :])
        else:
            aoff2 = pl.multiple_of(s * nsl, nsl)
            pltpu.sync_copy(acc.at[pl.ds(aoff2, nsl)],
                            out_hbm.at[t, pl.ds(aoff2, nsl), :])

        plsc.subcore_barrier()


def _edge_round(lat_t, a_t, b_t, from2d, to2d, vid2d=None, fuse_seg=False,
                lat_base=0):
    if fuse_seg:
        out_type = jax.ShapeDtypeStruct((TH, BSZ, D), jnp.float32)
    else:
        out_type = jax.ShapeDtypeStruct((TH, N, D), jnp.float32)
    vid_in = (vid2d.reshape(NS, VWT, SW) if fuse_seg
              else jnp.zeros((NS, VWT, SW), jnp.int32))
    zeros_in = jnp.zeros((BSZ, D), jnp.float32)

    def body(lat_hbm, a_hbm, b_hbm, from_hbm, to_hbm, vid_hbm, zeros_hbm,
             o_hbm, *scratch):
        if fuse_seg:
            _edge_body_fused(True, lat_base, lat_hbm, a_hbm, b_hbm,
                             from_hbm, to_hbm, vid_hbm, zeros_hbm,
                             None, o_hbm, *scratch)
        else:
            _edge_body_fused(False, lat_base, lat_hbm, a_hbm, b_hbm,
                             from_hbm, to_hbm, vid_hbm, zeros_hbm,
                             o_hbm, None, *scratch)

    fn = pl.kernel(
        body,
        out_type=out_type,
        mesh=plsc.VectorSubcoreMesh(**_MESH),
    scratch_types=[
            pltpu.VMEM_SHARED((N, D), jnp.float32),
            pltpu.VMEM_SHARED((BSZ, D), jnp.float32),
            pltpu.VMEM((WPT, EW), jnp.int32),
            pltpu.VMEM((WPT, EW), jnp.int32),
            pltpu.VMEM((VWT, SW), jnp.int32),
            pltpu.VMEM((EW, D), jnp.float32),
            pltpu.VMEM((EW, D), jnp.float32),
            pltpu.VMEM((EW, D), jnp.float32),
            pltpu.VMEM((EW, D), jnp.float32),
            pltpu.VMEM((EW, D), jnp.float32),
            pltpu.VMEM((EW, D), jnp.float32),
            pltpu.VMEM((EW, D), jnp.float32),
            pltpu.VMEM((EW, D), jnp.float32),
            pltpu.SemaphoreType.DMA,
            pltpu.SemaphoreType.DMA,
            pltpu.SemaphoreType.DMA,
            pltpu.SemaphoreType.DMA,
            pltpu.SemaphoreType.DMA,
            pltpu.SemaphoreType.DMA,
            pltpu.SemaphoreType.DMA,
            pltpu.SemaphoreType.DMA,
        ],
    )
    return fn(lat_t, a_t, b_t, from2d.reshape(NS, WPT, EW),
              to2d.reshape(NS, WPT, EW), vid_in, zeros_in)


def kernel(view_id, message_from, message_to, latents, ch, batch_size,
           W_m0, b_m0, W_m1, b_m1, W_r0, b_r0, W_r1, b_r1):
    ch_static = message_from.shape[0] // latents.shape[0] + 1
    inv = 1.0 / (ch_static - 1)

    vid2d = view_id.astype(jnp.int32).reshape(VW, SW)
    from2d = message_from.astype(jnp.int32).reshape(NWIN, EW)
    to2d = message_to.astype(jnp.int32).reshape(NWIN, EW)

    wa0 = W_m0[:D] * inv
    wb0 = W_m0[D:] * inv
    bv0 = (b_m0 * inv).reshape(1, D)
    wa1 = W_m1[:D] * inv
    wb1 = W_m1[D:] * inv
    bv1 = (b_m1 * inv).reshape(1, D)
    w0s = W_r0 * (1.0 / ch_static)
    b0v = b_r0.reshape(1, D)
    b1v = b_r1.reshape(1, D)

    lat0 = jnp.transpose(latents, (2, 0, 1))
    a0a, b0a = _ab(lat0, wa0, wb0, bv0, t_base=0)
    a0b, b0b = _ab(lat0, wa0, wb0, bv0, t_base=TH)
    lat1a = _edge_round(lat0, a0a, b0a, from2d, to2d, lat_base=0)
    lat1b = _edge_round(lat0, a0b, b0b, from2d, to2d, lat_base=TH)
    a1a, b1a = _ab(lat1a, wa1, wb1, bv1)
    a1b, b1b = _ab(lat1b, wa1, wb1, bv1)
    ya = _edge_round(lat1a, a1a, b1a, from2d, to2d, vid2d, fuse_seg=True)
    yb = _edge_round(lat1b, a1b, b1b, from2d, to2d, vid2d, fuse_seg=True)
    return _readout(ya, yb, w0s, b0v, W_r1, b1v)


# speedup vs baseline: 1.0160x; 1.0160x over previous
import jax
import jax.numpy as jnp
from jax import lax
from jax.experimental import pallas as pl
from jax.experimental.pallas import tpu as pltpu
from jax.experimental.pallas import tpu_sc as plsc

N = 8192
D = 128
T = 8
E = 24576
EW = 32
SW = 32
NWIN = E // EW
NC = 2
NS = 16
TH = T // 2
TPC = TH // NC
WPT = NWIN // NS
BSZ = 2048
VW = N // SW
VWT = VW // NS


def _ab_body(x_ref, wa_ref, wb_ref, bv_ref, a_ref, b_ref):
    x = x_ref[0]
    a_ref[0] = jnp.dot(x, wa_ref[...], preferred_element_type=jnp.float32)
    b_ref[0] = (jnp.dot(x, wb_ref[...], preferred_element_type=jnp.float32)
                + bv_ref[...])


def _ab(lat_t, wa, wb, bv, t_base=0):
    bn = 2048
    return pl.pallas_call(
        _ab_body,
        grid=(TH, N // bn),
        in_specs=[
            pl.BlockSpec((1, bn, D), lambda t, i: (t + t_base, i, 0)),
            pl.BlockSpec((D, D), lambda t, i: (0, 0)),
            pl.BlockSpec((D, D), lambda t, i: (0, 0)),
            pl.BlockSpec((1, D), lambda t, i: (0, 0)),
        ],
        out_specs=[
            pl.BlockSpec((1, bn, D), lambda t, i: (t, i, 0)),
            pl.BlockSpec((1, bn, D), lambda t, i: (t, i, 0)),
        ],
        out_shape=[
            jax.ShapeDtypeStruct((TH, N, D), jnp.float32),
            jax.ShapeDtypeStruct((TH, N, D), jnp.float32),
        ],
    )(lat_t, wa, wb, bv)


def _readout_body(ya_ref, yb_ref, w0_ref, b0_ref, w1_ref, b1_ref, o_ref):
    for t in range(T):
        y = ya_ref[t] if t < TH else yb_ref[t - TH]
        h = jnp.maximum(
            jnp.dot(y, w0_ref[...], preferred_element_type=jnp.float32)
            + b0_ref[...], 0.0)
        o_ref[:, t, :] = (
            jnp.dot(h, w1_ref[...], preferred_element_type=jnp.float32)
            + b1_ref[...])


def _readout(ya, yb, w0, b0, w1, b1):
    bb = 512
    return pl.pallas_call(
        _readout_body,
        grid=(BSZ // bb,),
        in_specs=[
            pl.BlockSpec((TH, bb, D), lambda i: (0, i, 0)),
            pl.BlockSpec((TH, bb, D), lambda i: (0, i, 0)),
            pl.BlockSpec((D, D), lambda i: (0, 0)),
            pl.BlockSpec((1, D), lambda i: (0, 0)),
            pl.BlockSpec((D, D), lambda i: (0, 0)),
            pl.BlockSpec((1, D), lambda i: (0, 0)),
        ],
        out_specs=pl.BlockSpec((bb, T, D), lambda i: (i, 0, 0)),
        out_shape=jax.ShapeDtypeStruct((BSZ, T, D), jnp.float32),
    )(ya, yb, w0, b0, w1, b1)


_MESH = dict(core_axis_name="c", subcore_axis_name="s", num_cores=NC,
             num_subcores=NS)


def _compute_msg(abuf, bbuf, mbuf):
    def row_body(r, carry):
        for g in range(D // 16):
            sl = pl.ds(g * 16, 16)
            mbuf[r, sl] = jnp.maximum(abuf[r, sl] + bbuf[r, sl], 0.0)
        return carry

    lax.fori_loop(0, EW, row_body, 0, unroll=4)


def _make_issue(a_t, b_t, idxf_all, idxt_all, abufs, bbufs, semas, sembs):
    def issue(w, bi):
        pltpu.async_copy(a_t.at[idxf_all.at[w]], abufs[bi], semas[bi])
        pltpu.async_copy(b_t.at[idxt_all.at[w]], bbufs[bi], sembs[bi])
    return issue


def _edge_windows(t, s, a_hbm, b_hbm, idxf_all, idxt_all, acc,
                  abufs, bbufs, semas, sembs, aux):
    a_t = a_hbm.at[t]
    b_t = b_hbm.at[t]
    mbufs, semss = aux
    issue = _make_issue(a_t, b_t, idxf_all, idxt_all, abufs, bbufs,
                        semas, sembs)

    def wait_gather(bi):
        pltpu.make_async_copy(a_t.at[idxf_all.at[0]], abufs[bi],
                              semas[bi]).wait()
        pltpu.make_async_copy(b_t.at[idxt_all.at[0]], bbufs[bi],
                              sembs[bi]).wait()

    def wait_scatter(mi):
        pltpu.make_async_copy(mbufs[mi], acc.at[idxt_all.at[0]],
                              semss[mi]).wait()

    npair = WPT // 2

    def do_window(w, bi, k):
        wait_gather(bi)

        @pl.when(k > 0)
        def _drain():
            wait_scatter(bi)

        _compute_msg(abufs[bi], bbufs[bi], mbufs[bi])
        pltpu.async_copy(mbufs[bi], acc.at[idxt_all.at[w]], semss[bi],
                         add=True)

    def pair_body(k, carry):
        w0 = 2 * k
        issue(w0 + 1, 1)
        do_window(w0, 0, k)

        @pl.when(k < npair - 1)
        def _prefetch():
            issue(w0 + 2, 0)

        do_window(w0 + 1, 1, k)
        return carry

    lax.fori_loop(0, npair, pair_body, 0)
    wait_scatter(0)
    wait_scatter(1)


def _edge_body_fused(fuse_seg, lat_base, lat_hbm, a_hbm, b_hbm,
                     from_hbm, to_hbm, vid_hbm, zeros_hbm, out_hbm, y_hbm,
                     acc, ysum, idxf_all, idxt_all, idxv,
                     ab0, ab1, bb0, bb1, mb0, mb1,
                     sema0, sema1, semb0, semb1, sems0, sems1):
    c = lax.axis_index("c")
    s = lax.axis_index("s")
    abufs = (ab0, ab1)
    bbufs = (bb0, bb1)
    semas = (sema0, sema1)
    sembs = (semb0, semb1)
    aux = ((mb0, mb1), (sems0, sems1))

    pltpu.sync_copy(from_hbm.at[s], idxf_all)
    pltpu.sync_copy(to_hbm.at[s], idxt_all)

    nsl = N // NS
    ysl = BSZ // NS
    for p in range(TPC):
        t = c * TPC + p
        tg = t + lat_base

        issue0 = _make_issue(a_hbm.at[t], b_hbm.at[t], idxf_all, idxt_all,
                             abufs, bbufs, semas, sembs)
        issue0(0, 0)

        aoff = pl.multiple_of(s * nsl, nsl)
        pltpu.sync_copy(lat_hbm.at[tg, pl.ds(aoff, nsl), :],
                        acc.at[pl.ds(aoff, nsl)])
        if fuse_seg:
            yoff = pl.multiple_of(s * ysl, ysl)
            pltpu.sync_copy(zeros_hbm.at[pl.ds(yoff, ysl)],
                            ysum.at[pl.ds(yoff, ysl)])

        plsc.subcore_barrier()

        _edge_windows(t, s, a_hbm, b_hbm, idxf_all, idxt_all, acc,
                      abufs, bbufs, semas, sembs, aux)
        plsc.subcore_barrier()

        if fuse_seg:
            pltpu.sync_copy(vid_hbm.at[s], idxv)
            for w in range(VWT):
                row = s * VWT + w
                off = pl.multiple_of(row * SW, SW)
                pltpu.sync_copy(acc.at[pl.ds(off, SW)], mb0)
                pltpu.sync_copy(mb0, ysum.at[idxv.at[w]], add=True)
            plsc.subcore_barrier()

        if fuse_seg:
            yoff2 = pl.multiple_of(s * ysl, ysl)
            pltpu.sync_copy(ysum.at[pl.ds(yoff2, ysl)],
                            y_hbm.at[t, pl.ds(yoff2, ysl), :])
        else:
            aoff2 = pl.multiple_of(s * nsl, nsl)
            pltpu.sync_copy(acc.at[pl.ds(aoff2, nsl)],
                            out_hbm.at[t, pl.ds(aoff2, nsl), :])

        plsc.subcore_barrier()


def _edge_round(lat_t, a_t, b_t, from2d, to2d, vid2d=None, fuse_seg=False,
                lat_base=0):
    if fuse_seg:
        out_type = jax.ShapeDtypeStruct((TH, BSZ, D), jnp.float32)
    else:
        out_type = jax.ShapeDtypeStruct((TH, N, D), jnp.float32)
    vid_in = (vid2d.reshape(NS, VWT, SW) if fuse_seg
              else jnp.zeros((NS, VWT, SW), jnp.int32))
    zeros_in = jnp.zeros((BSZ, D), jnp.float32)

    def body(lat_hbm, a_hbm, b_hbm, from_hbm, to_hbm, vid_hbm, zeros_hbm,
             o_hbm, *scratch):
        if fuse_seg:
            _edge_body_fused(True, lat_base, lat_hbm, a_hbm, b_hbm,
                             from_hbm, to_hbm, vid_hbm, zeros_hbm,
                             None, o_hbm, *scratch)
        else:
            _edge_body_fused(False, lat_base, lat_hbm, a_hbm, b_hbm,
                             from_hbm, to_hbm, vid_hbm, zeros_hbm,
                             o_hbm, None, *scratch)

    fn = pl.kernel(
        body,
        out_type=out_type,
        mesh=plsc.VectorSubcoreMesh(**_MESH),
    scratch_types=[
            pltpu.VMEM_SHARED((N, D), jnp.float32),
            pltpu.VMEM_SHARED((BSZ, D), jnp.float32),
            pltpu.VMEM((WPT, EW), jnp.int32),
            pltpu.VMEM((WPT, EW), jnp.int32),
            pltpu.VMEM((VWT, SW), jnp.int32),
            pltpu.VMEM((EW, D), jnp.float32),
            pltpu.VMEM((EW, D), jnp.float32),
            pltpu.VMEM((EW, D), jnp.float32),
            pltpu.VMEM((EW, D), jnp.float32),
            pltpu.VMEM((EW, D), jnp.float32),
            pltpu.VMEM((EW, D), jnp.float32),
            pltpu.SemaphoreType.DMA,
            pltpu.SemaphoreType.DMA,
            pltpu.SemaphoreType.DMA,
            pltpu.SemaphoreType.DMA,
            pltpu.SemaphoreType.DMA,
            pltpu.SemaphoreType.DMA,
        ],
    )
    return fn(lat_t, a_t, b_t, from2d.reshape(NS, WPT, EW),
              to2d.reshape(NS, WPT, EW), vid_in, zeros_in)


def kernel(view_id, message_from, message_to, latents, ch, batch_size,
           W_m0, b_m0, W_m1, b_m1, W_r0, b_r0, W_r1, b_r1):
    ch_static = message_from.shape[0] // latents.shape[0] + 1
    inv = 1.0 / (ch_static - 1)

    vid2d = view_id.astype(jnp.int32).reshape(VW, SW)
    from2d = message_from.astype(jnp.int32).reshape(NWIN, EW)
    to2d = message_to.astype(jnp.int32).reshape(NWIN, EW)

    wa0 = W_m0[:D] * inv
    wb0 = W_m0[D:] * inv
    bv0 = (b_m0 * inv).reshape(1, D)
    wa1 = W_m1[:D] * inv
    wb1 = W_m1[D:] * inv
    bv1 = (b_m1 * inv).reshape(1, D)
    w0s = W_r0 * (1.0 / ch_static)
    b0v = b_r0.reshape(1, D)
    b1v = b_r1.reshape(1, D)

    lat0 = jnp.transpose(latents, (2, 0, 1))
    a0a, b0a = _ab(lat0, wa0, wb0, bv0, t_base=0)
    a0b, b0b = _ab(lat0, wa0, wb0, bv0, t_base=TH)
    lat1a = _edge_round(lat0, a0a, b0a, from2d, to2d, lat_base=0)
    lat1b = _edge_round(lat0, a0b, b0b, from2d, to2d, lat_base=TH)
    a1a, b1a = _ab(lat1a, wa1, wb1, bv1)
    a1b, b1b = _ab(lat1b, wa1, wb1, bv1)
    ya = _edge_round(lat1a, a1a, b1a, from2d, to2d, vid2d, fuse_seg=True)
    yb = _edge_round(lat1b, a1b, b1b, from2d, to2d, vid2d, fuse_seg=True)
    return _readout(ya, yb, w0s, b0v, W_r1, b1v)
